# baseline (speedup 1.0000x reference)
import functools

import jax
import jax.numpy as jnp
from jax import lax
from jax.experimental import pallas as pl
from jax.experimental.pallas import tpu as pltpu
from jax.experimental.pallas import tpu_sc as plsc

_F32 = jnp.float32
_HI = lax.Precision.HIGHEST

_NC = 2
_NS = 16


def _dotT(a, b):
    return lax.dot_general(a, b, (((1,), (1,)), ((), ())),
                           preferred_element_type=_F32, precision=_HI)


def _k1_body(obj_ref, ws_ref, bs_ref, wo_ref, bo_ref, hs_ref, ho_ref, om_ref):
    obj = obj_ref[...]
    n = obj.shape[0]
    hs_ref[...] = _dotT(obj, ws_ref[...]) + bs_ref[...]
    ho_ref[...] = _dotT(obj, wo_ref[...]) + bo_ref[...]
    g = _dotT(obj, obj)
    s1 = jnp.sum(obj * obj, axis=1)
    sq = s1[:, None] + s1[None, :] - 2.0 * g
    om = jnp.where(sq < 0.25, 4.0, jnp.where(sq < 1.0, 1.0 / sq, 0.0))
    ii = lax.broadcasted_iota(jnp.int32, (n, n), 0)
    jj = lax.broadcasted_iota(jnp.int32, (n, n), 1)
    om_ref[...] = jnp.where(ii == jj, 0.0, om)


def _k1(obj, Ws, bs2, Wo, bo2):
    n, d = obj.shape
    return pl.pallas_call(
        _k1_body,
        out_shape=[jax.ShapeDtypeStruct((n, d), _F32),
                   jax.ShapeDtypeStruct((n, d), _F32),
                   jax.ShapeDtypeStruct((n, n), _F32)],
    )(obj, Ws, bs2, Wo, bo2)


def _k2(hs, ho, src, dst):
    n, d = hs.shape
    e = src.shape[0]
    nw = _NC * _NS
    e_per_w = e // nw
    chunk = 64
    nchunk = e_per_w // chunk
    mesh = plsc.VectorSubcoreMesh(core_axis_name="c", subcore_axis_name="s",
                                  num_cores=_NC, num_subcores=_NS)

    @functools.partial(
        pl.kernel,
        out_type=[jax.ShapeDtypeStruct((e, d), _F32),
                  jax.ShapeDtypeStruct((e, d), _F32)],
        mesh=mesh,
        scratch_types=[
            pltpu.VMEM((chunk,), jnp.int32),
            pltpu.VMEM((chunk, d), _F32),
            pltpu.SemaphoreType.DMA,
        ],
    )
    def k2(hs_hbm, ho_hbm, src_hbm, dst_hbm, hsg_hbm, hog_hbm,
           idx_v, rows_v, sem):
        wid = lax.axis_index("s") * _NC + lax.axis_index("c")
        for tab, idxarr, out in ((hs_hbm, src_hbm, hsg_hbm),
                                 (ho_hbm, dst_hbm, hog_hbm)):
            for c in range(nchunk):
                base = wid * e_per_w + c * chunk
                pltpu.sync_copy(idxarr.at[pl.ds(base, chunk)], idx_v)
                pltpu.async_copy(tab.at[idx_v], rows_v, sem).wait()
                pltpu.sync_copy(rows_v, out.at[pl.ds(base, chunk)])

    return k2(hs, ho, src, dst)


def _k3_body(hsg_ref, hog_ref, un_ref, ww_ref, bw_ref, flat_ref, out_ref):
    m = hsg_ref[...] * hog_ref[...] * un_ref[...]
    be, p = out_ref.shape[0], ww_ref.shape[0]
    af = _dotT(m, ww_ref[...]) + bw_ref[...]
    flat = flat_ref[0, 0, :]
    af16 = jnp.broadcast_to(af[:, None, :], (be, 16, p)).reshape(be, 16 * p)
    lane = lax.broadcasted_iota(jnp.int32, (be, 16 * p), 1)
    sel = (lane // p) == (flat % 16)[:, None]
    out_ref[...] = jnp.where(sel, af16, 0.0)


def _k3(hsg, hog, union, Ww, bw2, flat3):
    e, d = hsg.shape
    p = Ww.shape[0]
    be = 1024
    grid = (e // be,)
    row_spec = pl.BlockSpec((be, d), lambda i: (i, 0))
    return pl.pallas_call(
        _k3_body,
        grid=grid,
        in_specs=[row_spec, row_spec, row_spec,
                  pl.BlockSpec((p, d), lambda i: (0, 0)),
                  pl.BlockSpec((1, p), lambda i: (0, 0)),
                  pl.BlockSpec((1, 1, be), lambda i: (i, 0, 0))],
        out_specs=pl.BlockSpec((be, 16 * p), lambda i: (i, 0)),
        out_shape=jax.ShapeDtypeStruct((e, 16 * p), _F32),
    )(hsg, hog, union, Ww, bw2, flat3)


def _k4(af128, src, dst, zrows, n):
    e = af128.shape[0]
    npass = 2
    region = n * n // (npass * _NC)
    r16 = region // 16
    ept = e // _NS
    rpt = r16 // _NS
    ng = ept // 128
    mesh = plsc.VectorSubcoreMesh(core_axis_name="c", subcore_axis_name="s",
                                  num_cores=_NC, num_subcores=_NS)

    @functools.partial(
        pl.kernel,
        out_type=jax.ShapeDtypeStruct((n * n // 16, 128), _F32),
        mesh=mesh,
        scratch_types=[
            pltpu.VMEM((128, 128), _F32),
            pltpu.VMEM((ept,), jnp.int32),
            pltpu.VMEM((ept,), jnp.int32),
            pltpu.VMEM((ng, 128), jnp.int32),
            pltpu.VMEM_SHARED((r16 + 1, 128), _F32),
            pltpu.SemaphoreType.DMA,
        ],
    )
    def k4(af_hbm, src_hbm, dst_hbm, z_hbm, out_hbm,
           vals_v, src_v, dst_v, idx_v, acc_sh, sem):
        c = lax.axis_index("c")
        s = lax.axis_index("s")
        ebase = s * ept
        pltpu.sync_copy(src_hbm.at[pl.ds(ebase, ept)], src_v)
        pltpu.sync_copy(dst_hbm.at[pl.ds(ebase, ept)], dst_v)
        for q in range(npass):
            lo = (q * _NC + c) * region
            pltpu.sync_copy(z_hbm, acc_sh.at[pl.ds(s * rpt, rpt)])
            for k in range(ept // 16):
                s16 = src_v[pl.ds(k * 16, 16)]
                d16 = dst_v[pl.ds(k * 16, 16)]
                flat = s16 * n + d16
                inh = (flat >= lo) & (flat < lo + region)
                row = lax.shift_right_arithmetic(flat - lo, 4)
                idx_v[k // 8, pl.ds((k % 8) * 16, 16)] = jnp.where(inh, row, r16)
            plsc.subcore_barrier()
            for g in range(ng):
                pltpu.sync_copy(af_hbm.at[pl.ds(ebase + g * 128, 128)], vals_v)
                pltpu.sync_copy(vals_v, acc_sh.at[idx_v.at[g]], add=True)
            plsc.subcore_barrier()
            pltpu.sync_copy(acc_sh.at[pl.ds(s * rpt, rpt)],
                            out_hbm.at[pl.ds((q * _NC + c) * r16 + s * rpt, rpt)])
            if q + 1 < npass:
                plsc.subcore_barrier()

    return k4(af128, src, dst, zrows)


def _k5_body(a_ref, om_ref, out_ref):
    bi = a_ref.shape[0]
    n = a_ref.shape[1]
    p = a_ref.shape[2]
    ib = pl.program_id(0)
    xt = jnp.transpose(a_ref[...], (0, 2, 1))
    i0 = lax.broadcasted_iota(jnp.int32, (bi, p, n), 0)
    j2 = lax.broadcasted_iota(jnp.int32, (bi, p, n), 2)
    xt = xt - 10000.0 * (j2 == i0 + ib * bi).astype(_F32)
    mx = jnp.max(xt, axis=2, keepdims=True)
    ex = jnp.exp(xt - mx)
    sm = ex / jnp.sum(ex, axis=2, keepdims=True)
    res = sm * om_ref[...][:, None, :]
    out_ref[...] = jnp.transpose(res, (0, 2, 1))


def _k5(a3, om):
    n, _, p = a3.shape
    bi = 8
    return pl.pallas_call(
        _k5_body,
        grid=(n // bi,),
        in_specs=[pl.BlockSpec((bi, n, p), lambda i: (i, 0, 0)),
                  pl.BlockSpec((bi, n), lambda i: (i, 0))],
        out_specs=pl.BlockSpec((bi, n, p), lambda i: (i, 0, 0)),
        out_shape=jax.ShapeDtypeStruct((n, n, p), _F32),
    )(a3, om)


def kernel(obj_feats, union_feats, pair_idxs, Ws, bs, Wo, bo, Ww, bw):
    n, d = obj_feats.shape
    e = union_feats.shape[0]
    p = Ww.shape[0]
    src = pair_idxs[:, 0].astype(jnp.int32)
    dst = pair_idxs[:, 1].astype(jnp.int32)
    hs, ho, om = _k1(obj_feats, Ws, bs[None, :], Wo, bo[None, :])
    hsg, hog = _k2(hs, ho, src, dst)
    flat3 = (src * n + dst).reshape(e // 1024, 1, 1024)
    af128 = _k3(hsg, hog, union_feats, Ww, bw[None, :], flat3)
    zrows = jnp.zeros((n * n // 16 // (2 * _NC) // _NS, 128), _F32)
    a = _k4(af128, src, dst, zrows, n)
    return _k5(a.reshape(n, n, p), om)
